# baseline (speedup 1.0000x reference)
import functools

import jax
import jax.numpy as jnp
from jax import lax
from jax.experimental import pallas as pl
from jax.experimental.pallas import tpu as pltpu
from jax.experimental.pallas import tpu_sc as plsc


def _proj_body(h_ref, wsrc_ref, wdst_ref, hsrc_out, hdst_out):
    x = h_ref[...]
    dn = (((1,), (1,)), ((), ()))
    hsrc_out[...] = lax.dot_general(x, wsrc_ref[...], dn,
                                    preferred_element_type=jnp.float32)
    hdst_out[...] = lax.dot_general(x, wdst_ref[...], dn,
                                    preferred_element_type=jnp.float32)


def _project(h, W_src, W_dst):
    n, d_in = h.shape
    d_out = W_src.shape[0]
    blk = 1000
    grid = n // blk
    return pl.pallas_call(
        _proj_body,
        grid=(grid,),
        in_specs=[
            pl.BlockSpec((blk, d_in), lambda i: (i, 0)),
            pl.BlockSpec((d_out, d_in), lambda i: (0, 0)),
            pl.BlockSpec((d_out, d_in), lambda i: (0, 0)),
        ],
        out_specs=[
            pl.BlockSpec((blk, d_out), lambda i: (i, 0)),
            pl.BlockSpec((blk, d_out), lambda i: (i, 0)),
        ],
        out_shape=[
            jax.ShapeDtypeStruct((n, d_out), jnp.float32),
            jax.ShapeDtypeStruct((n, d_out), jnp.float32),
        ],
    )(h, W_src, W_dst)


def _make_edge_kernel(e_total, d, epw, chunk, nc, ns, n_nodes):
    nchunk = epw // chunk
    nsets = 4
    skew_add = 1
    skew_out = 2
    look = 2
    idxbytes = chunk * 4
    mesh = plsc.VectorSubcoreMesh(core_axis_name="c", subcore_axis_name="s")

    scratch = (
        [pltpu.VMEM((chunk, d), jnp.float32)] * nsets
        + [pltpu.VMEM((chunk,), jnp.int32)] * (2 * nsets)
        + [pltpu.SemaphoreType.DMA] * (4 * nsets)
        + [pltpu.VMEM_SHARED((n_nodes, d), jnp.float32)]
    )

    @functools.partial(
        pl.kernel,
        out_type=jax.ShapeDtypeStruct((e_total, d), jnp.float32),
        mesh=mesh,
        scratch_types=scratch,
    )
    def edge_kernel(hsrc_hbm, hdst_hbm, src_hbm, dst_hbm, out_hbm, *scr):
        rows = scr[0:nsets]
        idxs = scr[nsets:2 * nsets]
        idxd = scr[2 * nsets:3 * nsets]
        sem_idx = scr[3 * nsets:4 * nsets]
        sem_src = scr[4 * nsets:5 * nsets]
        sem_add = scr[5 * nsets:6 * nsets]
        sem_out = scr[6 * nsets:7 * nsets]
        src_tab = scr[7 * nsets]

        sid = lax.axis_index("s")
        wid = sid * nc + lax.axis_index("c")
        base = wid * epw

        def fetch_idx(c, si):
            pltpu.async_copy(src_hbm.at[pl.ds(base + c * chunk, chunk)],
                             idxs[si], sem_idx[si])
            pltpu.async_copy(dst_hbm.at[pl.ds(base + c * chunk, chunk)],
                             idxd[si], sem_idx[si])

        npr = (n_nodes // ns) // 8 * 8
        tail = n_nodes - npr * ns
        cp_tab = pltpu.async_copy(
            hsrc_hbm.at[pl.ds(sid * npr, npr)],
            src_tab.at[pl.ds(sid * npr, npr)], sem_add[0])
        for f in range(min(look, nchunk)):
            fetch_idx(f, f % nsets)
        if tail:
            @pl.when(sid == 0)
            def _():
                pltpu.sync_copy(hsrc_hbm.at[pl.ds(npr * ns, tail)],
                                src_tab.at[pl.ds(npr * ns, tail)])
        cp_tab.wait()
        plsc.subcore_barrier()

        def drain(buf, sem):
            pltpu.make_async_copy(hsrc_hbm.at[pl.ds(0, chunk)], buf, sem).wait()

        def drain_idx(si):
            pltpu.make_async_copy(src_hbm.at[pl.ds(0, chunk)], idxs[si],
                                  sem_idx[si]).wait()
            pltpu.make_async_copy(src_hbm.at[pl.ds(0, chunk)], idxd[si],
                                  sem_idx[si]).wait()

        def round_body(g, carry):
            for s in range(nsets):
                j = g * nsets + s
                t = (s - skew_add) % nsets
                u = (s - skew_out) % nsets
                f = (s + look) % nsets

                @pl.when((j >= skew_out) & (j < nchunk + skew_out))
                def _():
                    drain(rows[u], sem_add[u])
                    pltpu.async_copy(
                        rows[u],
                        out_hbm.at[
                            pl.ds(base + (j - skew_out) * chunk, chunk)],
                        sem_out[u])

                @pl.when(j + look < nchunk)
                def _():
                    fetch_idx(j + look, f)

                @pl.when(j < nchunk)
                def _():
                    @pl.when(j >= nsets)
                    def _():
                        drain(rows[s], sem_out[s])
                    drain_idx(s)
                    pltpu.async_copy(
                        src_tab.at[idxs[s]], rows[s], sem_src[s])

                @pl.when((j >= skew_add) & (j < nchunk + skew_add))
                def _():
                    drain(rows[t], sem_src[t])
                    pltpu.async_copy(
                        hdst_hbm.at[idxd[t]], rows[t], sem_add[t], add=True)
            return carry

        nrounds = (nchunk + skew_out + nsets - 1) // nsets + 1
        lax.fori_loop(0, nrounds, round_body, 0, unroll=False)
        for s in range(nsets):
            drain(rows[s], sem_out[s])

    return edge_kernel


def kernel(h, edge_index, W_src, W_dst):
    n, d_in = h.shape
    e_total = edge_index.shape[1]
    d = W_src.shape[0]

    h_src, h_dst = _project(h, W_src, W_dst)

    info = plsc.get_sparse_core_info()
    nc, ns = info.num_cores, info.num_subcores
    nw = nc * ns
    epw = e_total // nw
    chunk = 80

    src = edge_index[0]
    dst = edge_index[1]

    edge_kernel = _make_edge_kernel(e_total, d, epw, chunk, nc, ns, n)
    return edge_kernel(h_src, h_dst, src, dst)


# speedup vs baseline: 1.0011x; 1.0011x over previous
import functools

import jax
import jax.numpy as jnp
from jax import lax
from jax.experimental import pallas as pl
from jax.experimental.pallas import tpu as pltpu
from jax.experimental.pallas import tpu_sc as plsc


def _proj_body(h_ref, wsrc_ref, wdst_ref, hsrc_out, hdst_out):
    x = h_ref[...]
    dn = (((1,), (1,)), ((), ()))
    hsrc_out[...] = lax.dot_general(x, wsrc_ref[...], dn,
                                    preferred_element_type=jnp.float32)
    hdst_out[...] = lax.dot_general(x, wdst_ref[...], dn,
                                    preferred_element_type=jnp.float32)


def _project(h, W_src, W_dst):
    n, d_in = h.shape
    d_out = W_src.shape[0]
    blk = 1000
    grid = n // blk
    return pl.pallas_call(
        _proj_body,
        grid=(grid,),
        in_specs=[
            pl.BlockSpec((blk, d_in), lambda i: (i, 0)),
            pl.BlockSpec((d_out, d_in), lambda i: (0, 0)),
            pl.BlockSpec((d_out, d_in), lambda i: (0, 0)),
        ],
        out_specs=[
            pl.BlockSpec((blk, d_out), lambda i: (i, 0)),
            pl.BlockSpec((blk, d_out), lambda i: (i, 0)),
        ],
        out_shape=[
            jax.ShapeDtypeStruct((n, d_out), jnp.float32),
            jax.ShapeDtypeStruct((n, d_out), jnp.float32),
        ],
    )(h, W_src, W_dst)


def _make_edge_kernel(e_total, d, epw, chunk, nc, ns, n_nodes):
    nchunk = epw // chunk
    nsets = 4
    skew_add = 1
    skew_out = 2
    look = 2
    idxbytes = chunk * 4
    mesh = plsc.VectorSubcoreMesh(core_axis_name="c", subcore_axis_name="s")

    scratch = (
        [pltpu.VMEM((chunk, d), jnp.float32)] * nsets
        + [pltpu.VMEM((chunk,), jnp.int32)] * (2 * nsets)
        + [pltpu.SemaphoreType.DMA] * (4 * nsets)
        + [pltpu.VMEM_SHARED((n_nodes, d), jnp.float32)]
    )

    @functools.partial(
        pl.kernel,
        out_type=jax.ShapeDtypeStruct((e_total, d), jnp.float32),
        mesh=mesh,
        scratch_types=scratch,
    )
    def edge_kernel(hsrc_hbm, hdst_hbm, src_hbm, dst_hbm, out_hbm, *scr):
        rows = scr[0:nsets]
        idxs = scr[nsets:2 * nsets]
        idxd = scr[2 * nsets:3 * nsets]
        sem_idx = scr[3 * nsets:4 * nsets]
        sem_src = scr[4 * nsets:5 * nsets]
        sem_add = scr[5 * nsets:6 * nsets]
        sem_out = scr[6 * nsets:7 * nsets]
        src_tab = scr[7 * nsets]

        sid = lax.axis_index("s")
        wid = sid * nc + lax.axis_index("c")
        base = wid * epw
        npr = (n_nodes // ns) // 8 * 8
        tail = n_nodes - npr * ns
        cp_tab = pltpu.async_copy(
            hsrc_hbm.at[pl.ds(sid * npr, npr)],
            src_tab.at[pl.ds(sid * npr, npr)], sem_add[0])
        if tail:
            @pl.when(sid == 0)
            def _():
                pltpu.sync_copy(hsrc_hbm.at[pl.ds(npr * ns, tail)],
                                src_tab.at[pl.ds(npr * ns, tail)])
        cp_tab.wait()
        plsc.subcore_barrier()

        def fetch_idx(c, si):
            pltpu.async_copy(src_hbm.at[pl.ds(base + c * chunk, chunk)],
                             idxs[si], sem_idx[si])
            pltpu.async_copy(dst_hbm.at[pl.ds(base + c * chunk, chunk)],
                             idxd[si], sem_idx[si])

        def drain(buf, sem):
            pltpu.make_async_copy(hsrc_hbm.at[pl.ds(0, chunk)], buf, sem).wait()

        def drain_idx(si):
            pltpu.make_async_copy(src_hbm.at[pl.ds(0, chunk)], idxs[si],
                                  sem_idx[si]).wait()
            pltpu.make_async_copy(src_hbm.at[pl.ds(0, chunk)], idxd[si],
                                  sem_idx[si]).wait()

        for f in range(min(look, nchunk)):
            fetch_idx(f, f % nsets)

        def round_body(g, carry):
            for s in range(nsets):
                j = g * nsets + s
                t = (s - skew_add) % nsets
                u = (s - skew_out) % nsets
                f = (s + look) % nsets

                @pl.when((j >= skew_out) & (j < nchunk + skew_out))
                def _():
                    drain(rows[u], sem_add[u])
                    pltpu.async_copy(
                        rows[u],
                        out_hbm.at[
                            pl.ds(base + (j - skew_out) * chunk, chunk)],
                        sem_out[u])

                @pl.when(j + look < nchunk)
                def _():
                    fetch_idx(j + look, f)

                @pl.when(j < nchunk)
                def _():
                    @pl.when(j >= nsets)
                    def _():
                        drain(rows[s], sem_out[s])
                    drain_idx(s)
                    pltpu.async_copy(
                        src_tab.at[idxs[s]], rows[s], sem_src[s])

                @pl.when((j >= skew_add) & (j < nchunk + skew_add))
                def _():
                    drain(rows[t], sem_src[t])
                    pltpu.async_copy(
                        hdst_hbm.at[idxd[t]], rows[t], sem_add[t], add=True)
            return carry

        nrounds = (nchunk + skew_out + nsets - 1) // nsets + 1
        lax.fori_loop(0, nrounds, round_body, 0, unroll=False)
        for s in range(nsets):
            drain(rows[s], sem_out[s])

    return edge_kernel


def kernel(h, edge_index, W_src, W_dst):
    n, d_in = h.shape
    e_total = edge_index.shape[1]
    d = W_src.shape[0]

    h_src, h_dst = _project(h, W_src, W_dst)

    info = plsc.get_sparse_core_info()
    nc, ns = info.num_cores, info.num_subcores
    nw = nc * ns
    epw = e_total // nw
    chunk = 80

    src = edge_index[0]
    dst = edge_index[1]

    edge_kernel = _make_edge_kernel(e_total, d, epw, chunk, nc, ns, n)
    return edge_kernel(h_src, h_dst, src, dst)


# speedup vs baseline: 1.1348x; 1.1336x over previous
import functools

import jax
import jax.numpy as jnp
from jax import lax
from jax.experimental import pallas as pl
from jax.experimental.pallas import tpu as pltpu
from jax.experimental.pallas import tpu_sc as plsc


def _proj_body(h_ref, wsrc_ref, wdst_ref, hsrc_out, hdst_out):
    x = h_ref[...]
    dn = (((1,), (1,)), ((), ()))
    hsrc_out[...] = lax.dot_general(x, wsrc_ref[...], dn,
                                    preferred_element_type=jnp.float32)
    hdst_out[...] = lax.dot_general(x, wdst_ref[...], dn,
                                    preferred_element_type=jnp.float32)


def _project(h, W_src, W_dst):
    n, d_in = h.shape
    d_out = W_src.shape[0]
    blk = 1000
    grid = n // blk
    return pl.pallas_call(
        _proj_body,
        grid=(grid,),
        in_specs=[
            pl.BlockSpec((blk, d_in), lambda i: (i, 0)),
            pl.BlockSpec((d_out, d_in), lambda i: (0, 0)),
            pl.BlockSpec((d_out, d_in), lambda i: (0, 0)),
        ],
        out_specs=[
            pl.BlockSpec((blk, d_out), lambda i: (i, 0)),
            pl.BlockSpec((blk, d_out), lambda i: (i, 0)),
        ],
        out_shape=[
            jax.ShapeDtypeStruct((n, d_out), jnp.float32),
            jax.ShapeDtypeStruct((n, d_out), jnp.float32),
        ],
    )(h, W_src, W_dst)


def _make_edge_kernel(e_total, d, chunk, nc, ns, n_nodes):
    nw = nc * ns
    tchunks = e_total // chunk
    nchunk_lo = tchunks // nw
    rem = tchunks - nchunk_lo * nw
    nchunk_hi = nchunk_lo + (1 if rem else 0)
    nr = 3
    ni = 4
    unroll = 12
    skew_add = 1
    skew_out = 2
    look = 2
    mesh = plsc.VectorSubcoreMesh(core_axis_name="c", subcore_axis_name="s")

    scratch = (
        [pltpu.VMEM((chunk, d), jnp.float32)] * nr
        + [pltpu.VMEM((chunk,), jnp.int32)] * (2 * ni)
        + [pltpu.SemaphoreType.DMA] * (ni + 3 * nr)
        + [pltpu.VMEM_SHARED((n_nodes, d), jnp.float32)]
    )

    @functools.partial(
        pl.kernel,
        out_type=jax.ShapeDtypeStruct((e_total, d), jnp.float32),
        mesh=mesh,
        scratch_types=scratch,
    )
    def edge_kernel(hsrc_hbm, hdst_hbm, src_hbm, dst_hbm, out_hbm, *scr):
        rows = scr[0:nr]
        idxs = scr[nr:nr + ni]
        idxd = scr[nr + ni:nr + 2 * ni]
        o = nr + 2 * ni
        sem_idx = scr[o:o + ni]
        sem_src = scr[o + ni:o + ni + nr]
        sem_add = scr[o + ni + nr:o + ni + 2 * nr]
        sem_out = scr[o + ni + 2 * nr:o + ni + 3 * nr]
        src_tab = scr[o + ni + 3 * nr]

        sid = lax.axis_index("s")
        wid = sid * nc + lax.axis_index("c")
        is_hi = wid < rem
        nchunk = lax.select(is_hi, jnp.int32(nchunk_hi), jnp.int32(nchunk_lo))
        base = chunk * lax.select(
            is_hi, wid * nchunk_hi,
            rem * nchunk_hi + (wid - rem) * nchunk_lo)
        npr = (n_nodes // ns) // 8 * 8
        tail = n_nodes - npr * ns
        cp_tab = pltpu.async_copy(
            hsrc_hbm.at[pl.ds(sid * npr, npr)],
            src_tab.at[pl.ds(sid * npr, npr)], sem_add[0])
        if tail:
            @pl.when(sid == 0)
            def _():
                pltpu.sync_copy(hsrc_hbm.at[pl.ds(npr * ns, tail)],
                                src_tab.at[pl.ds(npr * ns, tail)])
        cp_tab.wait()
        plsc.subcore_barrier()

        def fetch_idx(c, si):
            pltpu.async_copy(src_hbm.at[pl.ds(base + c * chunk, chunk)],
                             idxs[si], sem_idx[si])
            pltpu.async_copy(dst_hbm.at[pl.ds(base + c * chunk, chunk)],
                             idxd[si], sem_idx[si])

        def drain(buf, sem):
            pltpu.make_async_copy(hsrc_hbm.at[pl.ds(0, chunk)], buf, sem).wait()

        def drain_idx(si):
            pltpu.make_async_copy(src_hbm.at[pl.ds(0, chunk)], idxs[si],
                                  sem_idx[si]).wait()
            pltpu.make_async_copy(src_hbm.at[pl.ds(0, chunk)], idxd[si],
                                  sem_idx[si]).wait()

        for f in range(min(look, nchunk_lo)):
            fetch_idx(f, f % ni)

        def round_body(g, carry):
            for s in range(unroll):
                j = g * unroll + s
                sr = s % nr
                si = s % ni
                t = (s - skew_add) % nr
                ti = (s - skew_add) % ni
                u = (s - skew_out) % nr
                fi = (s + look) % ni

                @pl.when((j >= skew_out) & (j < nchunk + skew_out))
                def _():
                    drain(rows[u], sem_add[u])
                    pltpu.async_copy(
                        rows[u],
                        out_hbm.at[
                            pl.ds(base + (j - skew_out) * chunk, chunk)],
                        sem_out[u])

                @pl.when(j + look < nchunk)
                def _():
                    fetch_idx(j + look, fi)

                @pl.when(j < nchunk)
                def _():
                    @pl.when(j >= nr)
                    def _():
                        drain(rows[sr], sem_out[sr])
                    drain_idx(si)
                    pltpu.async_copy(
                        src_tab.at[idxs[si]], rows[sr], sem_src[sr])

                @pl.when((j >= skew_add) & (j < nchunk + skew_add))
                def _():
                    drain(rows[t], sem_src[t])
                    pltpu.async_copy(
                        hdst_hbm.at[idxd[ti]], rows[t], sem_add[t], add=True)
            return carry

        nrounds = (nchunk_hi + skew_out + unroll - 1) // unroll + 1
        lax.fori_loop(0, nrounds, round_body, 0, unroll=False)
        for s in range(nr):
            drain(rows[s], sem_out[s])
    return edge_kernel


def kernel(h, edge_index, W_src, W_dst):
    n, d_in = h.shape
    e_total = edge_index.shape[1]
    d = W_src.shape[0]

    h_src, h_dst = _project(h, W_src, W_dst)

    info = plsc.get_sparse_core_info()
    nc, ns = info.num_cores, info.num_subcores
    chunk = 128 if e_total % 128 == 0 else 80

    src = edge_index[0]
    dst = edge_index[1]

    edge_kernel = _make_edge_kernel(e_total, d, chunk, nc, ns, n)
    return edge_kernel(h_src, h_dst, src, dst)


# speedup vs baseline: 1.1611x; 1.0231x over previous
import functools

import jax
import jax.numpy as jnp
from jax import lax
from jax.experimental import pallas as pl
from jax.experimental.pallas import tpu as pltpu
from jax.experimental.pallas import tpu_sc as plsc


def _proj_body(h_ref, wsrc_ref, wdst_ref, hsrc_out, hdst_out):
    x = h_ref[...]
    dn = (((1,), (1,)), ((), ()))
    hsrc_out[...] = lax.dot_general(x, wsrc_ref[...], dn,
                                    preferred_element_type=jnp.float32)
    hdst_out[...] = lax.dot_general(x, wdst_ref[...], dn,
                                    preferred_element_type=jnp.float32)


def _project(h, W_src, W_dst):
    n, d_in = h.shape
    d_out = W_src.shape[0]
    blk = n
    grid = n // blk
    return pl.pallas_call(
        _proj_body,
        grid=(grid,),
        in_specs=[
            pl.BlockSpec((blk, d_in), lambda i: (i, 0)),
            pl.BlockSpec((d_out, d_in), lambda i: (0, 0)),
            pl.BlockSpec((d_out, d_in), lambda i: (0, 0)),
        ],
        out_specs=[
            pl.BlockSpec((blk, d_out), lambda i: (i, 0)),
            pl.BlockSpec((blk, d_out), lambda i: (i, 0)),
        ],
        out_shape=[
            jax.ShapeDtypeStruct((n, d_out), jnp.float32),
            jax.ShapeDtypeStruct((n, d_out), jnp.float32),
        ],
    )(h, W_src, W_dst)


def _make_edge_kernel(e_total, d, chunk, nc, ns, n_nodes):
    nw = nc * ns
    tchunks = e_total // chunk
    nchunk_lo = tchunks // nw
    rem = tchunks - nchunk_lo * nw
    nchunk_hi = nchunk_lo + (1 if rem else 0)
    nr = 3
    ni = 4
    unroll = 12
    skew_add = 1
    skew_out = 2
    look = 2
    mesh = plsc.VectorSubcoreMesh(core_axis_name="c", subcore_axis_name="s")

    scratch = (
        [pltpu.VMEM((chunk, d), jnp.float32)] * nr
        + [pltpu.VMEM((chunk,), jnp.int32)] * (2 * ni)
        + [pltpu.SemaphoreType.DMA] * (ni + 3 * nr)
        + [pltpu.VMEM_SHARED((n_nodes, d), jnp.float32)]
    )

    @functools.partial(
        pl.kernel,
        out_type=jax.ShapeDtypeStruct((e_total, d), jnp.float32),
        mesh=mesh,
        scratch_types=scratch,
    )
    def edge_kernel(hsrc_hbm, hdst_hbm, src_hbm, dst_hbm, out_hbm, *scr):
        rows = scr[0:nr]
        idxs = scr[nr:nr + ni]
        idxd = scr[nr + ni:nr + 2 * ni]
        o = nr + 2 * ni
        sem_idx = scr[o:o + ni]
        sem_src = scr[o + ni:o + ni + nr]
        sem_add = scr[o + ni + nr:o + ni + 2 * nr]
        sem_out = scr[o + ni + 2 * nr:o + ni + 3 * nr]
        src_tab = scr[o + ni + 3 * nr]

        sid = lax.axis_index("s")
        wid = sid * nc + lax.axis_index("c")
        is_hi = wid < rem
        nchunk = lax.select(is_hi, jnp.int32(nchunk_hi), jnp.int32(nchunk_lo))
        base = chunk * lax.select(
            is_hi, wid * nchunk_hi,
            rem * nchunk_hi + (wid - rem) * nchunk_lo)
        npr = (n_nodes // ns) // 8 * 8
        tail = n_nodes - npr * ns
        cp_tab = pltpu.async_copy(
            hsrc_hbm.at[pl.ds(sid * npr, npr)],
            src_tab.at[pl.ds(sid * npr, npr)], sem_add[0])
        if tail:
            @pl.when(sid == 0)
            def _():
                pltpu.sync_copy(hsrc_hbm.at[pl.ds(npr * ns, tail)],
                                src_tab.at[pl.ds(npr * ns, tail)])
        cp_tab.wait()
        plsc.subcore_barrier()

        def fetch_idx(c, si):
            pltpu.async_copy(src_hbm.at[pl.ds(base + c * chunk, chunk)],
                             idxs[si], sem_idx[si])
            pltpu.async_copy(dst_hbm.at[pl.ds(base + c * chunk, chunk)],
                             idxd[si], sem_idx[si])

        def drain(buf, sem):
            pltpu.make_async_copy(hsrc_hbm.at[pl.ds(0, chunk)], buf, sem).wait()

        def drain_idx(si):
            pltpu.make_async_copy(src_hbm.at[pl.ds(0, chunk)], idxs[si],
                                  sem_idx[si]).wait()
            pltpu.make_async_copy(src_hbm.at[pl.ds(0, chunk)], idxd[si],
                                  sem_idx[si]).wait()

        for f in range(min(look, nchunk_lo)):
            fetch_idx(f, f % ni)

        def round_body(g, carry):
            for s in range(unroll):
                j = g * unroll + s
                sr = s % nr
                si = s % ni
                t = (s - skew_add) % nr
                ti = (s - skew_add) % ni
                u = (s - skew_out) % nr
                fi = (s + look) % ni

                @pl.when((j >= skew_out) & (j < nchunk + skew_out))
                def _():
                    drain(rows[u], sem_add[u])
                    pltpu.async_copy(
                        rows[u],
                        out_hbm.at[
                            pl.ds(base + (j - skew_out) * chunk, chunk)],
                        sem_out[u])

                @pl.when(j + look < nchunk)
                def _():
                    fetch_idx(j + look, fi)

                @pl.when(j < nchunk)
                def _():
                    @pl.when(j >= nr)
                    def _():
                        drain(rows[sr], sem_out[sr])
                    drain_idx(si)
                    pltpu.async_copy(
                        src_tab.at[idxs[si]], rows[sr], sem_src[sr])

                @pl.when((j >= skew_add) & (j < nchunk + skew_add))
                def _():
                    drain(rows[t], sem_src[t])
                    pltpu.async_copy(
                        hdst_hbm.at[idxd[ti]], rows[t], sem_add[t], add=True)
            return carry

        nrounds = (nchunk_hi + skew_out + unroll - 1) // unroll + 1
        lax.fori_loop(0, nrounds, round_body, 0, unroll=False)
        for s in range(nr):
            drain(rows[s], sem_out[s])
    return edge_kernel


def kernel(h, edge_index, W_src, W_dst):
    n, d_in = h.shape
    e_total = edge_index.shape[1]
    d = W_src.shape[0]

    h_src, h_dst = _project(h, W_src, W_dst)

    info = plsc.get_sparse_core_info()
    nc, ns = info.num_cores, info.num_subcores
    chunk = 128 if e_total % 128 == 0 else 80

    src = edge_index[0]
    dst = edge_index[1]

    edge_kernel = _make_edge_kernel(e_total, d, chunk, nc, ns, n)
    return edge_kernel(h_src, h_dst, src, dst)
